# Initial kernel scaffold; baseline (speedup 1.0000x reference)
import jax
import jax.numpy as jnp
from jax.experimental import pallas as pl


def kernel(x, e, gamma, beta, W1e, b1e, root1, bias1, W2e, b2e, root2, bias2, Wf, bf, Wa, ba, Wd, bd, edge_index, i):
    raise NotImplementedError("write your pallas kernel here")


# speedup vs baseline: 3.7784x; 3.7784x over previous
import functools

import jax
import jax.numpy as jnp
from jax import lax
from jax.experimental import pallas as pl
from jax.experimental.pallas import tpu as pltpu
from jax.experimental.pallas import tpu_sc as plsc

F32 = jnp.float32
N_GRAPHS = 256
IDXBLK = 128


def _pre_body(x_ref, scale_ref, beta_ref, root1_ref, bias1_ref, r1_ref):
    xn = x_ref[...] * scale_ref[...] + beta_ref[...]
    r1_ref[...] = jnp.dot(xn, root1_ref[...], preferred_element_type=F32) + bias1_ref[...]


def _msg_body(e_ref, xs_ref, v_ref, scale_ref, beta_ref, out_ref, *, h, d_edge):
    xs = xs_ref[...] * scale_ref[...] + beta_ref[...]
    p = jnp.dot(xs, v_ref[...], preferred_element_type=F32)
    acc = p[:, d_edge * h:]
    for d in range(d_edge):
        acc = acc + e_ref[:, d:d + 1] * p[:, d * h:(d + 1) * h]
    out_ref[...] = acc


def _hidden_body(agg_ref, r1_ref, root2_ref, bias2_ref, h1_ref, r2_ref):
    h1 = jnp.maximum(agg_ref[0] + agg_ref[1] + r1_ref[...], 0.0)
    h1_ref[...] = h1
    r2_ref[...] = jnp.dot(h1, root2_ref[...], preferred_element_type=F32) + bias2_ref[...]


def _pool_body(agg_ref, r2_ref, wf_ref, bf_ref, wa_ref, ba_ref, wd_ref, bd_ref,
               seg_ref, out_ref, acc_ref, *, n_graphs, nblocks):
    j = pl.program_id(0)
    h2 = jnp.maximum(agg_ref[0] + agg_ref[1] + r2_ref[...], 0.0)
    feat = jnp.dot(h2, wf_ref[...], preferred_element_type=F32) + bf_ref[...]
    attn = jax.nn.sigmoid(jnp.dot(h2, wa_ref[...], preferred_element_type=F32) + ba_ref[...])
    p = feat * attn
    seg = seg_ref[...]
    onehot = (seg == lax.broadcasted_iota(jnp.int32, (n_graphs, seg.shape[1]), 0)).astype(F32)
    part = jnp.dot(onehot, p, preferred_element_type=F32)

    @pl.when(j == 0)
    def _():
        acc_ref[...] = part

    @pl.when(j > 0)
    def _():
        acc_ref[...] = acc_ref[...] + part

    @pl.when(j == nblocks - 1)
    def _():
        out_ref[...] = (jnp.dot(acc_ref[...], wd_ref[...], preferred_element_type=F32)
                        + bd_ref[...])


def _sc_gather(table, idx2d, feat_dim):
    nrows_idx, _ = idx2d.shape
    e_total = nrows_idx * IDXBLK
    info = plsc.get_sparse_core_info()
    nc, ns = info.num_cores, info.num_subcores
    nw = nc * ns
    chunk = e_total // nw
    kblk = chunk // IDXBLK
    mesh = plsc.VectorSubcoreMesh(core_axis_name="c", subcore_axis_name="s")

    @functools.partial(
        pl.kernel,
        out_type=jax.ShapeDtypeStruct((e_total, feat_dim), F32),
        mesh=mesh,
        compiler_params=pltpu.CompilerParams(use_tc_tiling_on_sc=False),
        scratch_types=[
            pltpu.VMEM((kblk, IDXBLK), jnp.int32),
            pltpu.VMEM((chunk, feat_dim), F32),
            pltpu.SemaphoreType.DMA,
        ],
    )
    def gk(table_hbm, idx_hbm, out_hbm, idx_v, rows_v, sem):
        c = lax.axis_index("c")
        s = lax.axis_index("s")
        w = s * nc + c
        pltpu.sync_copy(idx_hbm.at[pl.ds(w * kblk, kblk)], idx_v)
        copies = []
        for j in range(kblk):
            copies.append(pltpu.async_copy(
                table_hbm.at[idx_v.at[j]],
                rows_v.at[pl.ds(j * IDXBLK, IDXBLK)], sem))
        for cp in copies:
            cp.wait()
        pltpu.sync_copy(rows_v, out_hbm.at[pl.ds(w * chunk, chunk)])

    return gk(table, idx2d)


def _sc_scatter_add(vals, idx2d, zeros_nh, n_nodes, feat_dim):
    nrows_idx, _ = idx2d.shape
    e_total = nrows_idx * IDXBLK
    info = plsc.get_sparse_core_info()
    nc, ns = info.num_cores, info.num_subcores
    nw = nc * ns
    chunk = e_total // nw
    kblk = chunk // IDXBLK
    rows_per_tile = n_nodes // ns
    mesh = plsc.VectorSubcoreMesh(core_axis_name="c", subcore_axis_name="s")

    @functools.partial(
        pl.kernel,
        out_type=jax.ShapeDtypeStruct((nc, n_nodes, feat_dim), F32),
        mesh=mesh,
        compiler_params=pltpu.CompilerParams(use_tc_tiling_on_sc=False),
        scratch_types=[
            pltpu.VMEM((kblk, IDXBLK), jnp.int32),
            pltpu.VMEM((chunk, feat_dim), F32),
            pltpu.VMEM_SHARED((n_nodes, feat_dim), F32),
            pltpu.SemaphoreType.DMA,
        ],
    )
    def sk(vals_hbm, idx_hbm, zeros_hbm, out_hbm, idx_v, vals_v, acc_sh, sem):
        c = lax.axis_index("c")
        s = lax.axis_index("s")
        w = s * nc + c
        r0 = s * rows_per_tile
        pltpu.sync_copy(zeros_hbm.at[pl.ds(r0, rows_per_tile)],
                        acc_sh.at[pl.ds(r0, rows_per_tile)])
        plsc.subcore_barrier()
        pltpu.sync_copy(idx_hbm.at[pl.ds(w * kblk, kblk)], idx_v)
        pltpu.sync_copy(vals_hbm.at[pl.ds(w * chunk, chunk)], vals_v)
        for j in range(kblk):
            pltpu.sync_copy(vals_v.at[pl.ds(j * IDXBLK, IDXBLK)],
                            acc_sh.at[idx_v.at[j]], add=True)
        plsc.subcore_barrier()
        pltpu.sync_copy(acc_sh.at[pl.ds(r0, rows_per_tile)],
                        out_hbm.at[c, pl.ds(r0, rows_per_tile)])

    return sk(vals, idx2d, zeros_nh)


def kernel(x, e, gamma, beta, W1e, b1e, root1, bias1, W2e, b2e, root2, bias2,
           Wf, bf, Wa, ba, Wd, bd, edge_index, i):
    n, f_in = x.shape
    e_total, d_edge = e.shape
    h = root1.shape[1]
    p_ch = Wf.shape[1]
    n_out = Wd.shape[1]

    scale = (gamma * lax.rsqrt(jnp.float32(1.0 + 1e-3))).reshape(1, f_in)
    beta2 = beta.reshape(1, f_in)
    src2d = edge_index[0].reshape(e_total // IDXBLK, IDXBLK)
    dst2d = edge_index[1].reshape(e_total // IDXBLK, IDXBLK)
    v1 = (jnp.concatenate([W1e, b1e[None, :]], axis=0)
          .reshape(d_edge + 1, f_in, h).transpose(1, 0, 2)
          .reshape(f_in, (d_edge + 1) * h))
    v2 = (jnp.concatenate([W2e, b2e[None, :]], axis=0)
          .reshape(d_edge + 1, h, h).transpose(1, 0, 2)
          .reshape(h, (d_edge + 1) * h))
    zeros_nh = jnp.zeros((n, h), F32)
    seg = i.reshape(1, n)

    r1 = pl.pallas_call(
        _pre_body,
        out_shape=jax.ShapeDtypeStruct((n, h), F32),
    )(x, scale, beta2, root1, bias1.reshape(1, h))

    xs = _sc_gather(x, src2d, f_in)

    blk = 2048
    nblk = e_total // blk
    m1 = pl.pallas_call(
        functools.partial(_msg_body, h=h, d_edge=d_edge),
        grid=(nblk,),
        in_specs=[pl.BlockSpec((blk, d_edge), lambda j: (j, 0)),
                  pl.BlockSpec((blk, f_in), lambda j: (j, 0)),
                  pl.BlockSpec((f_in, (d_edge + 1) * h), lambda j: (0, 0)),
                  pl.BlockSpec((1, f_in), lambda j: (0, 0)),
                  pl.BlockSpec((1, f_in), lambda j: (0, 0))],
        out_specs=pl.BlockSpec((blk, h), lambda j: (j, 0)),
        out_shape=jax.ShapeDtypeStruct((e_total, h), F32),
    )(e, xs, v1, scale, beta2)

    agg1 = _sc_scatter_add(m1, dst2d, zeros_nh, n, h)

    h1, r2 = pl.pallas_call(
        _hidden_body,
        out_shape=[jax.ShapeDtypeStruct((n, h), F32),
                   jax.ShapeDtypeStruct((n, h), F32)],
    )(agg1, r1, root2, bias2.reshape(1, h))

    h1s = _sc_gather(h1, src2d, h)

    ones = jnp.ones((1, h), F32)
    zeros1h = jnp.zeros((1, h), F32)
    m2 = pl.pallas_call(
        functools.partial(_msg_body, h=h, d_edge=d_edge),
        grid=(nblk,),
        in_specs=[pl.BlockSpec((blk, d_edge), lambda j: (j, 0)),
                  pl.BlockSpec((blk, h), lambda j: (j, 0)),
                  pl.BlockSpec((h, (d_edge + 1) * h), lambda j: (0, 0)),
                  pl.BlockSpec((1, h), lambda j: (0, 0)),
                  pl.BlockSpec((1, h), lambda j: (0, 0))],
        out_specs=pl.BlockSpec((blk, h), lambda j: (j, 0)),
        out_shape=jax.ShapeDtypeStruct((e_total, h), F32),
    )(e, h1s, v2, ones, zeros1h)

    agg2 = _sc_scatter_add(m2, dst2d, zeros_nh, n, h)

    nb = 8
    bn = n // nb
    out = pl.pallas_call(
        functools.partial(_pool_body, n_graphs=N_GRAPHS, nblocks=nb),
        grid=(nb,),
        in_specs=[pl.BlockSpec((2, bn, h), lambda j: (0, j, 0)),
                  pl.BlockSpec((bn, h), lambda j: (j, 0)),
                  pl.BlockSpec((h, p_ch), lambda j: (0, 0)),
                  pl.BlockSpec((1, p_ch), lambda j: (0, 0)),
                  pl.BlockSpec((h, p_ch), lambda j: (0, 0)),
                  pl.BlockSpec((1, p_ch), lambda j: (0, 0)),
                  pl.BlockSpec((p_ch, n_out), lambda j: (0, 0)),
                  pl.BlockSpec((1, n_out), lambda j: (0, 0)),
                  pl.BlockSpec((1, bn), lambda j: (0, j))],
        out_specs=pl.BlockSpec((N_GRAPHS, n_out), lambda j: (0, 0)),
        out_shape=jax.ShapeDtypeStruct((N_GRAPHS, n_out), F32),
        scratch_shapes=[pltpu.VMEM((N_GRAPHS, p_ch), F32)],
    )(agg2, r2, Wf, bf.reshape(1, p_ch), Wa, ba.reshape(1, p_ch),
      Wd, bd.reshape(1, n_out), seg)
    return out


# speedup vs baseline: 3.9427x; 1.0435x over previous
import functools

import jax
import jax.numpy as jnp
from jax import lax
from jax.experimental import pallas as pl
from jax.experimental.pallas import tpu as pltpu
from jax.experimental.pallas import tpu_sc as plsc

F32 = jnp.float32
N_GRAPHS = 256
IDXBLK = 128


def _pre_body(x_ref, scale_ref, beta_ref, root1_ref, bias1_ref, r1_ref):
    xn = x_ref[...] * scale_ref[...] + beta_ref[...]
    r1_ref[...] = jnp.dot(xn, root1_ref[...], preferred_element_type=F32) + bias1_ref[...]


def _msg_body(ea_ref, xs_ref, s_ref, r_ref, v_ref, scale_ref, beta_ref, out_ref):
    xn = xs_ref[...] * scale_ref[...] + beta_ref[...]
    z = (jnp.dot(ea_ref[...], s_ref[...], preferred_element_type=F32)
         * jnp.dot(xn, r_ref[...], preferred_element_type=F32))
    out_ref[...] = jnp.dot(z, v_ref[...], preferred_element_type=F32)


def _hidden_body(agg_ref, r1_ref, root2_ref, bias2_ref, h1_ref, r2_ref):
    h1 = jnp.maximum(agg_ref[0] + agg_ref[1] + r1_ref[...], 0.0)
    h1_ref[...] = h1
    r2_ref[...] = jnp.dot(h1, root2_ref[...], preferred_element_type=F32) + bias2_ref[...]


def _pool_body(agg_ref, r2_ref, wf_ref, bf_ref, wa_ref, ba_ref, wd_ref, bd_ref,
               seg_ref, out_ref, acc_ref, *, n_graphs, nblocks):
    j = pl.program_id(0)
    h2 = jnp.maximum(agg_ref[0] + agg_ref[1] + r2_ref[...], 0.0)
    feat = jnp.dot(h2, wf_ref[...], preferred_element_type=F32) + bf_ref[...]
    attn = jax.nn.sigmoid(jnp.dot(h2, wa_ref[...], preferred_element_type=F32) + ba_ref[...])
    p = feat * attn
    seg = seg_ref[...]
    onehot = (seg == lax.broadcasted_iota(jnp.int32, (n_graphs, seg.shape[1]), 0)).astype(F32)
    part = jnp.dot(onehot, p, preferred_element_type=F32)

    @pl.when(j == 0)
    def _():
        acc_ref[...] = part

    @pl.when(j > 0)
    def _():
        acc_ref[...] = acc_ref[...] + part

    @pl.when(j == nblocks - 1)
    def _():
        out_ref[...] = (jnp.dot(acc_ref[...], wd_ref[...], preferred_element_type=F32)
                        + bd_ref[...])


def _sc_gather(table, idx2d, feat_dim):
    nrows_idx, _ = idx2d.shape
    e_total = nrows_idx * IDXBLK
    info = plsc.get_sparse_core_info()
    nc, ns = info.num_cores, info.num_subcores
    nw = nc * ns
    chunk = e_total // nw
    kblk = chunk // IDXBLK
    mesh = plsc.VectorSubcoreMesh(core_axis_name="c", subcore_axis_name="s")

    @functools.partial(
        pl.kernel,
        out_type=jax.ShapeDtypeStruct((e_total, feat_dim), F32),
        mesh=mesh,
        compiler_params=pltpu.CompilerParams(use_tc_tiling_on_sc=False),
        scratch_types=[
            pltpu.VMEM((kblk, IDXBLK), jnp.int32),
            pltpu.VMEM((chunk, feat_dim), F32),
            pltpu.SemaphoreType.DMA,
        ],
    )
    def gk(table_hbm, idx_hbm, out_hbm, idx_v, rows_v, sem):
        c = lax.axis_index("c")
        s = lax.axis_index("s")
        w = s * nc + c
        pltpu.sync_copy(idx_hbm.at[pl.ds(w * kblk, kblk)], idx_v)
        copies = []
        for j in range(kblk):
            copies.append(pltpu.async_copy(
                table_hbm.at[idx_v.at[j]],
                rows_v.at[pl.ds(j * IDXBLK, IDXBLK)], sem))
        for cp in copies:
            cp.wait()
        pltpu.sync_copy(rows_v, out_hbm.at[pl.ds(w * chunk, chunk)])

    return gk(table, idx2d)


def _sc_scatter_add(vals, idx2d, zeros_nh, n_nodes, feat_dim):
    nrows_idx, _ = idx2d.shape
    e_total = nrows_idx * IDXBLK
    info = plsc.get_sparse_core_info()
    nc, ns = info.num_cores, info.num_subcores
    nw = nc * ns
    chunk = e_total // nw
    kblk = chunk // IDXBLK
    rows_per_tile = n_nodes // ns
    mesh = plsc.VectorSubcoreMesh(core_axis_name="c", subcore_axis_name="s")

    @functools.partial(
        pl.kernel,
        out_type=jax.ShapeDtypeStruct((nc, n_nodes, feat_dim), F32),
        mesh=mesh,
        compiler_params=pltpu.CompilerParams(use_tc_tiling_on_sc=False),
        scratch_types=[
            pltpu.VMEM((kblk, IDXBLK), jnp.int32),
            pltpu.VMEM((chunk, feat_dim), F32),
            pltpu.VMEM_SHARED((n_nodes, feat_dim), F32),
            pltpu.SemaphoreType.DMA,
        ],
    )
    def sk(vals_hbm, idx_hbm, zeros_hbm, out_hbm, idx_v, vals_v, acc_sh, sem):
        c = lax.axis_index("c")
        s = lax.axis_index("s")
        w = s * nc + c
        r0 = s * rows_per_tile
        pltpu.sync_copy(zeros_hbm.at[pl.ds(r0, rows_per_tile)],
                        acc_sh.at[pl.ds(r0, rows_per_tile)])
        plsc.subcore_barrier()
        pltpu.sync_copy(idx_hbm.at[pl.ds(w * kblk, kblk)], idx_v)
        pltpu.sync_copy(vals_hbm.at[pl.ds(w * chunk, chunk)], vals_v)
        for j in range(kblk):
            pltpu.sync_copy(vals_v.at[pl.ds(j * IDXBLK, IDXBLK)],
                            acc_sh.at[idx_v.at[j]], add=True)
        plsc.subcore_barrier()
        pltpu.sync_copy(acc_sh.at[pl.ds(r0, rows_per_tile)],
                        out_hbm.at[c, pl.ds(r0, rows_per_tile)])

    return sk(vals, idx2d, zeros_nh)


def kernel(x, e, gamma, beta, W1e, b1e, root1, bias1, W2e, b2e, root2, bias2,
           Wf, bf, Wa, ba, Wd, bd, edge_index, i):
    n, f_in = x.shape
    e_total, d_edge = e.shape
    h = root1.shape[1]
    p_ch = Wf.shape[1]
    n_out = Wd.shape[1]

    scale = (gamma * lax.rsqrt(jnp.float32(1.0 + 1e-3))).reshape(1, f_in)
    beta2 = beta.reshape(1, f_in)
    src2d = edge_index[0].reshape(e_total // IDXBLK, IDXBLK)
    dst2d = edge_index[1].reshape(e_total // IDXBLK, IDXBLK)
    dd = d_edge + 1
    ea = jnp.concatenate([e, jnp.ones((e_total, 1), F32)], axis=1)
    v1 = jnp.concatenate([W1e, b1e[None, :]], axis=0).reshape(dd * f_in, h)
    v2 = jnp.concatenate([W2e, b2e[None, :]], axis=0).reshape(dd * h, h)
    s1 = jnp.kron(jnp.eye(dd, dtype=F32), jnp.ones((1, f_in), F32))
    r1m = jnp.tile(jnp.eye(f_in, dtype=F32), (1, dd))
    s2 = jnp.kron(jnp.eye(dd, dtype=F32), jnp.ones((1, h), F32))
    r2m = jnp.tile(jnp.eye(h, dtype=F32), (1, dd))
    zeros_nh = jnp.zeros((n, h), F32)
    seg = i.reshape(1, n)

    r1 = pl.pallas_call(
        _pre_body,
        out_shape=jax.ShapeDtypeStruct((n, h), F32),
    )(x, scale, beta2, root1, bias1.reshape(1, h))

    xs = _sc_gather(x, src2d, f_in)

    blk = 2048
    nblk = e_total // blk
    m1 = pl.pallas_call(
        _msg_body,
        grid=(nblk,),
        in_specs=[pl.BlockSpec((blk, dd), lambda j: (j, 0)),
                  pl.BlockSpec((blk, f_in), lambda j: (j, 0)),
                  pl.BlockSpec((dd, dd * f_in), lambda j: (0, 0)),
                  pl.BlockSpec((f_in, dd * f_in), lambda j: (0, 0)),
                  pl.BlockSpec((dd * f_in, h), lambda j: (0, 0)),
                  pl.BlockSpec((1, f_in), lambda j: (0, 0)),
                  pl.BlockSpec((1, f_in), lambda j: (0, 0))],
        out_specs=pl.BlockSpec((blk, h), lambda j: (j, 0)),
        out_shape=jax.ShapeDtypeStruct((e_total, h), F32),
    )(ea, xs, s1, r1m, v1, scale, beta2)

    agg1 = _sc_scatter_add(m1, dst2d, zeros_nh, n, h)

    h1, r2 = pl.pallas_call(
        _hidden_body,
        out_shape=[jax.ShapeDtypeStruct((n, h), F32),
                   jax.ShapeDtypeStruct((n, h), F32)],
    )(agg1, r1, root2, bias2.reshape(1, h))

    h1s = _sc_gather(h1, src2d, h)

    ones = jnp.ones((1, h), F32)
    zeros1h = jnp.zeros((1, h), F32)
    m2 = pl.pallas_call(
        _msg_body,
        grid=(nblk,),
        in_specs=[pl.BlockSpec((blk, dd), lambda j: (j, 0)),
                  pl.BlockSpec((blk, h), lambda j: (j, 0)),
                  pl.BlockSpec((dd, dd * h), lambda j: (0, 0)),
                  pl.BlockSpec((h, dd * h), lambda j: (0, 0)),
                  pl.BlockSpec((dd * h, h), lambda j: (0, 0)),
                  pl.BlockSpec((1, h), lambda j: (0, 0)),
                  pl.BlockSpec((1, h), lambda j: (0, 0))],
        out_specs=pl.BlockSpec((blk, h), lambda j: (j, 0)),
        out_shape=jax.ShapeDtypeStruct((e_total, h), F32),
    )(ea, h1s, s2, r2m, v2, ones, zeros1h)

    agg2 = _sc_scatter_add(m2, dst2d, zeros_nh, n, h)

    nb = 8
    bn = n // nb
    out = pl.pallas_call(
        functools.partial(_pool_body, n_graphs=N_GRAPHS, nblocks=nb),
        grid=(nb,),
        in_specs=[pl.BlockSpec((2, bn, h), lambda j: (0, j, 0)),
                  pl.BlockSpec((bn, h), lambda j: (j, 0)),
                  pl.BlockSpec((h, p_ch), lambda j: (0, 0)),
                  pl.BlockSpec((1, p_ch), lambda j: (0, 0)),
                  pl.BlockSpec((h, p_ch), lambda j: (0, 0)),
                  pl.BlockSpec((1, p_ch), lambda j: (0, 0)),
                  pl.BlockSpec((p_ch, n_out), lambda j: (0, 0)),
                  pl.BlockSpec((1, n_out), lambda j: (0, 0)),
                  pl.BlockSpec((1, bn), lambda j: (0, j))],
        out_specs=pl.BlockSpec((N_GRAPHS, n_out), lambda j: (0, 0)),
        out_shape=jax.ShapeDtypeStruct((N_GRAPHS, n_out), F32),
        scratch_shapes=[pltpu.VMEM((N_GRAPHS, p_ch), F32)],
    )(agg2, r2, Wf, bf.reshape(1, p_ch), Wa, ba.reshape(1, p_ch),
      Wd, bd.reshape(1, n_out), seg)
    return out


# speedup vs baseline: 5.1404x; 1.3038x over previous
import functools

import jax
import jax.numpy as jnp
from jax import lax
from jax.experimental import pallas as pl
from jax.experimental.pallas import tpu as pltpu
from jax.experimental.pallas import tpu_sc as plsc

F32 = jnp.float32
N_GRAPHS = 256
IDXBLK = 128
LANES = 128


def _pre_body(x_ref, scale_ref, beta_ref, root1_ref, bias1_ref, xp_ref, r1_ref):
    xn = x_ref[...] * scale_ref[...] + beta_ref[...]
    xp_ref[...] = jnp.concatenate(
        [xn, jnp.zeros((xn.shape[0], LANES - xn.shape[1]), F32)], axis=1)
    r1_ref[...] = jnp.dot(xn, root1_ref[...], preferred_element_type=F32) + bias1_ref[...]


def _msg_body(ea_ref, xs_ref, s_ref, r_ref, v_ref, out_ref, *, f_in):
    xn = xs_ref[:, :f_in]
    e7 = lax.dot_general(ea_ref[...], s_ref[...], (((0,), (0,)), ((), ())),
                         preferred_element_type=F32)
    z = e7 * jnp.dot(xn, r_ref[...], preferred_element_type=F32)
    m = jnp.dot(z, v_ref[...], preferred_element_type=F32)
    out_ref[...] = jnp.concatenate(
        [m, jnp.zeros((m.shape[0], LANES - m.shape[1]), F32)], axis=1)


def _hidden_body(agg_ref, r1_ref, root2_ref, bias2_ref, h1_ref, r2_ref):
    h = r1_ref.shape[1]
    h1 = jnp.maximum(agg_ref[0][:, :h] + agg_ref[1][:, :h] + r1_ref[...], 0.0)
    h1_ref[...] = jnp.concatenate(
        [h1, jnp.zeros((h1.shape[0], LANES - h), F32)], axis=1)
    r2_ref[...] = jnp.dot(h1, root2_ref[...], preferred_element_type=F32) + bias2_ref[...]


def _pool_body(agg_ref, r2_ref, wf_ref, bf_ref, wa_ref, ba_ref, wd_ref, bd_ref,
               seg_ref, out_ref, acc_ref, *, n_graphs, nblocks):
    j = pl.program_id(0)
    h = r2_ref.shape[1]
    h2 = jnp.maximum(agg_ref[0][:, :h] + agg_ref[1][:, :h] + r2_ref[...], 0.0)
    feat = jnp.dot(h2, wf_ref[...], preferred_element_type=F32) + bf_ref[...]
    attn = jax.nn.sigmoid(jnp.dot(h2, wa_ref[...], preferred_element_type=F32) + ba_ref[...])
    p = feat * attn
    seg = seg_ref[...]
    onehot = (seg == lax.broadcasted_iota(jnp.int32, (n_graphs, seg.shape[1]), 0)).astype(F32)
    part = jnp.dot(onehot, p, preferred_element_type=F32)

    @pl.when(j == 0)
    def _():
        acc_ref[...] = part

    @pl.when(j > 0)
    def _():
        acc_ref[...] = acc_ref[...] + part

    @pl.when(j == nblocks - 1)
    def _():
        out_ref[...] = (jnp.dot(acc_ref[...], wd_ref[...], preferred_element_type=F32)
                        + bd_ref[...])


def _sc_gather(table, idx2d):
    nrows_idx, _ = idx2d.shape
    e_total = nrows_idx * IDXBLK
    info = plsc.get_sparse_core_info()
    nc, ns = info.num_cores, info.num_subcores
    nw = nc * ns
    chunk = e_total // nw
    kblk = chunk // IDXBLK
    half = chunk // 2
    khalf = kblk // 2
    mesh = plsc.VectorSubcoreMesh(core_axis_name="c", subcore_axis_name="s")

    @functools.partial(
        pl.kernel,
        out_type=jax.ShapeDtypeStruct((e_total, LANES), F32),
        mesh=mesh,
        scratch_types=[
            pltpu.VMEM((kblk, IDXBLK), jnp.int32),
            pltpu.VMEM((half, LANES), F32),
            pltpu.SemaphoreType.DMA,
        ],
    )
    def gk(table_hbm, idx_hbm, out_hbm, idx_v, rows_v, sem):
        c = lax.axis_index("c")
        s = lax.axis_index("s")
        w = s * nc + c
        pltpu.sync_copy(idx_hbm.at[pl.ds(w * kblk, kblk)], idx_v)
        for hf in range(2):
            copies = []
            for j in range(khalf):
                copies.append(pltpu.async_copy(
                    table_hbm.at[idx_v.at[hf * khalf + j]],
                    rows_v.at[pl.ds(j * IDXBLK, IDXBLK)], sem))
            for cp in copies:
                cp.wait()
            pltpu.sync_copy(rows_v, out_hbm.at[pl.ds(w * chunk + hf * half, half)])

    return gk(table, idx2d)


def _sc_scatter_add(vals, idx2d, n_nodes):
    nrows_idx, _ = idx2d.shape
    e_total = nrows_idx * IDXBLK
    info = plsc.get_sparse_core_info()
    nc, ns = info.num_cores, info.num_subcores
    nw = nc * ns
    chunk = e_total // nw
    kblk = chunk // IDXBLK
    nparts = 4
    part = chunk // nparts
    kpart = kblk // nparts
    rows_per_tile = n_nodes // ns
    mesh = plsc.VectorSubcoreMesh(core_axis_name="c", subcore_axis_name="s")

    zrows = 16

    @functools.partial(
        pl.kernel,
        out_type=jax.ShapeDtypeStruct((nc, n_nodes, LANES), F32),
        mesh=mesh,
        scratch_types=[
            pltpu.VMEM((kblk, IDXBLK), jnp.int32),
            pltpu.VMEM((part, LANES), F32),
            pltpu.VMEM((zrows, LANES), F32),
            pltpu.VMEM_SHARED((n_nodes, LANES), F32),
            pltpu.SemaphoreType.DMA,
        ],
    )
    def sk(vals_hbm, idx_hbm, out_hbm, idx_v, vals_v, zbuf, acc_sh, sem):
        c = lax.axis_index("c")
        s = lax.axis_index("s")
        w = s * nc + c
        r0 = s * rows_per_tile
        nlane16 = LANES // 16

        def bz(k, _):
            zbuf[k // nlane16, pl.ds((k % nlane16) * 16, 16)] = jnp.zeros((16,), F32)
            return 0

        lax.fori_loop(0, zrows * nlane16, bz, 0)
        for t in range(rows_per_tile // zrows):
            pltpu.sync_copy(zbuf, acc_sh.at[pl.ds(r0 + t * zrows, zrows)])
        plsc.subcore_barrier()
        pltpu.sync_copy(idx_hbm.at[pl.ds(w * kblk, kblk)], idx_v)
        for hf in range(nparts):
            pltpu.sync_copy(vals_hbm.at[pl.ds(w * chunk + hf * part, part)], vals_v)
            for j in range(kpart):
                pltpu.sync_copy(vals_v.at[pl.ds(j * IDXBLK, IDXBLK)],
                                acc_sh.at[idx_v.at[hf * kpart + j]], add=True)
        plsc.subcore_barrier()
        pltpu.sync_copy(acc_sh.at[pl.ds(r0, rows_per_tile)],
                        out_hbm.at[c, pl.ds(r0, rows_per_tile)])

    return sk(vals, idx2d)


def kernel(x, e, gamma, beta, W1e, b1e, root1, bias1, W2e, b2e, root2, bias2,
           Wf, bf, Wa, ba, Wd, bd, edge_index, i):
    n, f_in = x.shape
    e_total, d_edge = e.shape
    h = root1.shape[1]
    p_ch = Wf.shape[1]
    n_out = Wd.shape[1]

    scale = (gamma * lax.rsqrt(jnp.float32(1.0 + 1e-3))).reshape(1, f_in)
    beta2 = beta.reshape(1, f_in)
    src2d = edge_index[0].reshape(e_total // IDXBLK, IDXBLK)
    dst2d = edge_index[1].reshape(e_total // IDXBLK, IDXBLK)
    dd = d_edge + 1
    ea_t = jnp.concatenate([e.T, jnp.ones((1, e_total), F32)], axis=0)
    v1 = jnp.concatenate([W1e, b1e[None, :]], axis=0).reshape(dd * f_in, h)
    v2 = jnp.concatenate([W2e, b2e[None, :]], axis=0).reshape(dd * h, h)
    s1 = jnp.kron(jnp.eye(dd, dtype=F32), jnp.ones((1, f_in), F32))
    r1m = jnp.tile(jnp.eye(f_in, dtype=F32), (1, dd))
    s2 = jnp.kron(jnp.eye(dd, dtype=F32), jnp.ones((1, h), F32))
    r2m = jnp.tile(jnp.eye(h, dtype=F32), (1, dd))
    seg = i.reshape(1, n)

    xp, r1 = pl.pallas_call(
        _pre_body,
        out_shape=[jax.ShapeDtypeStruct((n, LANES), F32),
                   jax.ShapeDtypeStruct((n, h), F32)],
    )(x, scale, beta2, root1, bias1.reshape(1, h))

    xs = _sc_gather(xp, src2d)

    blk = 2048
    nblk = e_total // blk
    m1 = pl.pallas_call(
        functools.partial(_msg_body, f_in=f_in),
        grid=(nblk,),
        in_specs=[pl.BlockSpec((dd, blk), lambda j: (0, j)),
                  pl.BlockSpec((blk, LANES), lambda j: (j, 0)),
                  pl.BlockSpec((dd, dd * f_in), lambda j: (0, 0)),
                  pl.BlockSpec((f_in, dd * f_in), lambda j: (0, 0)),
                  pl.BlockSpec((dd * f_in, h), lambda j: (0, 0))],
        out_specs=pl.BlockSpec((blk, LANES), lambda j: (j, 0)),
        out_shape=jax.ShapeDtypeStruct((e_total, LANES), F32),
    )(ea_t, xs, s1, r1m, v1)

    agg1 = _sc_scatter_add(m1, dst2d, n)

    h1p, r2 = pl.pallas_call(
        _hidden_body,
        out_shape=[jax.ShapeDtypeStruct((n, LANES), F32),
                   jax.ShapeDtypeStruct((n, h), F32)],
    )(agg1, r1, root2, bias2.reshape(1, h))

    h1s = _sc_gather(h1p, src2d)

    m2 = pl.pallas_call(
        functools.partial(_msg_body, f_in=h),
        grid=(nblk,),
        in_specs=[pl.BlockSpec((dd, blk), lambda j: (0, j)),
                  pl.BlockSpec((blk, LANES), lambda j: (j, 0)),
                  pl.BlockSpec((dd, dd * h), lambda j: (0, 0)),
                  pl.BlockSpec((h, dd * h), lambda j: (0, 0)),
                  pl.BlockSpec((dd * h, h), lambda j: (0, 0))],
        out_specs=pl.BlockSpec((blk, LANES), lambda j: (j, 0)),
        out_shape=jax.ShapeDtypeStruct((e_total, LANES), F32),
    )(ea_t, h1s, s2, r2m, v2)

    agg2 = _sc_scatter_add(m2, dst2d, n)

    nb = 8
    bn = n // nb
    out = pl.pallas_call(
        functools.partial(_pool_body, n_graphs=N_GRAPHS, nblocks=nb),
        grid=(nb,),
        in_specs=[pl.BlockSpec((2, bn, LANES), lambda j: (0, j, 0)),
                  pl.BlockSpec((bn, h), lambda j: (j, 0)),
                  pl.BlockSpec((h, p_ch), lambda j: (0, 0)),
                  pl.BlockSpec((1, p_ch), lambda j: (0, 0)),
                  pl.BlockSpec((h, p_ch), lambda j: (0, 0)),
                  pl.BlockSpec((1, p_ch), lambda j: (0, 0)),
                  pl.BlockSpec((p_ch, n_out), lambda j: (0, 0)),
                  pl.BlockSpec((1, n_out), lambda j: (0, 0)),
                  pl.BlockSpec((1, bn), lambda j: (0, j))],
        out_specs=pl.BlockSpec((N_GRAPHS, n_out), lambda j: (0, 0)),
        out_shape=jax.ShapeDtypeStruct((N_GRAPHS, n_out), F32),
        scratch_shapes=[pltpu.VMEM((N_GRAPHS, p_ch), F32)],
    )(agg2, r2, Wf, bf.reshape(1, p_ch), Wa, ba.reshape(1, p_ch),
      Wd, bd.reshape(1, n_out), seg)
    return out


# speedup vs baseline: 5.3193x; 1.0348x over previous
import functools

import jax
import jax.numpy as jnp
from jax import lax
from jax.experimental import pallas as pl
from jax.experimental.pallas import tpu as pltpu
from jax.experimental.pallas import tpu_sc as plsc

F32 = jnp.float32
N_GRAPHS = 256
IDXBLK = 128
LANES = 128


def _pre_body(x_ref, scale_ref, beta_ref, root1_ref, bias1_ref, xp_ref, r1_ref):
    xn = x_ref[...] * scale_ref[...] + beta_ref[...]
    xp_ref[...] = jnp.concatenate(
        [xn, jnp.zeros((xn.shape[0], LANES - xn.shape[1]), F32)], axis=1)
    r1_ref[...] = jnp.dot(xn, root1_ref[...], preferred_element_type=F32) + bias1_ref[...]


def _msg_body(ea_ref, xs_ref, s_ref, r_ref, v_ref, out_ref, *, f_in):
    bf = jnp.bfloat16
    xn = xs_ref[:, :f_in].astype(bf)
    e7 = lax.dot_general(ea_ref[...].astype(bf), s_ref[...].astype(bf),
                         (((0,), (0,)), ((), ())),
                         preferred_element_type=F32)
    z = (e7 * jnp.dot(xn, r_ref[...].astype(bf), preferred_element_type=F32)).astype(bf)
    m = jnp.dot(z, v_ref[...].astype(bf), preferred_element_type=F32)
    out_ref[...] = jnp.concatenate(
        [m, jnp.zeros((m.shape[0], LANES - m.shape[1]), F32)], axis=1)


def _hidden_body(agg_ref, r1_ref, root2_ref, bias2_ref, h1_ref, r2_ref):
    h = r1_ref.shape[1]
    h1 = jnp.maximum(agg_ref[0][:, :h] + agg_ref[1][:, :h] + r1_ref[...], 0.0)
    h1_ref[...] = jnp.concatenate(
        [h1, jnp.zeros((h1.shape[0], LANES - h), F32)], axis=1)
    r2_ref[...] = jnp.dot(h1, root2_ref[...], preferred_element_type=F32) + bias2_ref[...]


def _pool_body(agg_ref, r2_ref, wf_ref, bf_ref, wa_ref, ba_ref, wd_ref, bd_ref,
               seg_ref, out_ref, acc_ref, *, n_graphs, nblocks):
    j = pl.program_id(0)
    h = r2_ref.shape[1]
    h2 = jnp.maximum(agg_ref[0][:, :h] + agg_ref[1][:, :h] + r2_ref[...], 0.0)
    feat = jnp.dot(h2, wf_ref[...], preferred_element_type=F32) + bf_ref[...]
    attn = jax.nn.sigmoid(jnp.dot(h2, wa_ref[...], preferred_element_type=F32) + ba_ref[...])
    p = feat * attn
    seg = seg_ref[...]
    onehot = (seg == lax.broadcasted_iota(jnp.int32, (n_graphs, seg.shape[1]), 0)).astype(F32)
    part = jnp.dot(onehot, p, preferred_element_type=F32)

    @pl.when(j == 0)
    def _():
        acc_ref[...] = part

    @pl.when(j > 0)
    def _():
        acc_ref[...] = acc_ref[...] + part

    @pl.when(j == nblocks - 1)
    def _():
        out_ref[...] = (jnp.dot(acc_ref[...], wd_ref[...], preferred_element_type=F32)
                        + bd_ref[...])


def _sc_gather(table, idx2d):
    nrows_idx, _ = idx2d.shape
    e_total = nrows_idx * IDXBLK
    info = plsc.get_sparse_core_info()
    nc, ns = info.num_cores, info.num_subcores
    nw = nc * ns
    chunk = e_total // nw
    kblk = chunk // IDXBLK
    half = chunk // 2
    khalf = kblk // 2
    mesh = plsc.VectorSubcoreMesh(core_axis_name="c", subcore_axis_name="s")

    @functools.partial(
        pl.kernel,
        out_type=jax.ShapeDtypeStruct((e_total, LANES), F32),
        mesh=mesh,
        scratch_types=[
            pltpu.VMEM((kblk, IDXBLK), jnp.int32),
            pltpu.VMEM((half, LANES), F32),
            pltpu.SemaphoreType.DMA,
        ],
    )
    def gk(table_hbm, idx_hbm, out_hbm, idx_v, rows_v, sem):
        c = lax.axis_index("c")
        s = lax.axis_index("s")
        w = s * nc + c
        pltpu.sync_copy(idx_hbm.at[pl.ds(w * kblk, kblk)], idx_v)
        for hf in range(2):
            copies = []
            for j in range(khalf):
                copies.append(pltpu.async_copy(
                    table_hbm.at[idx_v.at[hf * khalf + j]],
                    rows_v.at[pl.ds(j * IDXBLK, IDXBLK)], sem))
            for cp in copies:
                cp.wait()
            pltpu.sync_copy(rows_v, out_hbm.at[pl.ds(w * chunk + hf * half, half)])

    return gk(table, idx2d)


def _sc_scatter_add(vals, idx2d, n_nodes):
    nrows_idx, _ = idx2d.shape
    e_total = nrows_idx * IDXBLK
    info = plsc.get_sparse_core_info()
    nc, ns = info.num_cores, info.num_subcores
    nw = nc * ns
    chunk = e_total // nw
    kblk = chunk // IDXBLK
    nparts = 4
    part = chunk // nparts
    kpart = kblk // nparts
    rows_per_tile = n_nodes // ns
    mesh = plsc.VectorSubcoreMesh(core_axis_name="c", subcore_axis_name="s")

    zrows = 16

    @functools.partial(
        pl.kernel,
        out_type=jax.ShapeDtypeStruct((nc, n_nodes, LANES), F32),
        mesh=mesh,
        scratch_types=[
            pltpu.VMEM((kblk, IDXBLK), jnp.int32),
            pltpu.VMEM((part, LANES), F32),
            pltpu.VMEM((zrows, LANES), F32),
            pltpu.VMEM_SHARED((n_nodes, LANES), F32),
            pltpu.SemaphoreType.DMA,
        ],
    )
    def sk(vals_hbm, idx_hbm, out_hbm, idx_v, vals_v, zbuf, acc_sh, sem):
        c = lax.axis_index("c")
        s = lax.axis_index("s")
        w = s * nc + c
        r0 = s * rows_per_tile
        nlane16 = LANES // 16

        def bz(k, _):
            zbuf[k // nlane16, pl.ds((k % nlane16) * 16, 16)] = jnp.zeros((16,), F32)
            return 0

        lax.fori_loop(0, zrows * nlane16, bz, 0)
        for t in range(rows_per_tile // zrows):
            pltpu.sync_copy(zbuf, acc_sh.at[pl.ds(r0 + t * zrows, zrows)])
        plsc.subcore_barrier()
        pltpu.sync_copy(idx_hbm.at[pl.ds(w * kblk, kblk)], idx_v)
        for hf in range(nparts):
            pltpu.sync_copy(vals_hbm.at[pl.ds(w * chunk + hf * part, part)], vals_v)
            for j in range(kpart):
                pltpu.sync_copy(vals_v.at[pl.ds(j * IDXBLK, IDXBLK)],
                                acc_sh.at[idx_v.at[hf * kpart + j]], add=True)
        plsc.subcore_barrier()
        pltpu.sync_copy(acc_sh.at[pl.ds(r0, rows_per_tile)],
                        out_hbm.at[c, pl.ds(r0, rows_per_tile)])

    return sk(vals, idx2d)


def kernel(x, e, gamma, beta, W1e, b1e, root1, bias1, W2e, b2e, root2, bias2,
           Wf, bf, Wa, ba, Wd, bd, edge_index, i):
    n, f_in = x.shape
    e_total, d_edge = e.shape
    h = root1.shape[1]
    p_ch = Wf.shape[1]
    n_out = Wd.shape[1]

    scale = (gamma * lax.rsqrt(jnp.float32(1.0 + 1e-3))).reshape(1, f_in)
    beta2 = beta.reshape(1, f_in)
    src2d = edge_index[0].reshape(e_total // IDXBLK, IDXBLK)
    dst2d = edge_index[1].reshape(e_total // IDXBLK, IDXBLK)
    dd = d_edge + 1
    ea_t = jnp.concatenate([e.T, jnp.ones((1, e_total), F32)], axis=0)
    v1 = jnp.concatenate([W1e, b1e[None, :]], axis=0).reshape(dd * f_in, h)
    v2 = jnp.concatenate([W2e, b2e[None, :]], axis=0).reshape(dd * h, h)
    s1 = jnp.kron(jnp.eye(dd, dtype=F32), jnp.ones((1, f_in), F32))
    r1m = jnp.tile(jnp.eye(f_in, dtype=F32), (1, dd))
    s2 = jnp.kron(jnp.eye(dd, dtype=F32), jnp.ones((1, h), F32))
    r2m = jnp.tile(jnp.eye(h, dtype=F32), (1, dd))
    seg = i.reshape(1, n)

    xp, r1 = pl.pallas_call(
        _pre_body,
        out_shape=[jax.ShapeDtypeStruct((n, LANES), F32),
                   jax.ShapeDtypeStruct((n, h), F32)],
    )(x, scale, beta2, root1, bias1.reshape(1, h))

    xs = _sc_gather(xp, src2d)

    blk = 4096
    nblk = e_total // blk
    m1 = pl.pallas_call(
        functools.partial(_msg_body, f_in=f_in),
        grid=(nblk,),
        in_specs=[pl.BlockSpec((dd, blk), lambda j: (0, j)),
                  pl.BlockSpec((blk, LANES), lambda j: (j, 0)),
                  pl.BlockSpec((dd, dd * f_in), lambda j: (0, 0)),
                  pl.BlockSpec((f_in, dd * f_in), lambda j: (0, 0)),
                  pl.BlockSpec((dd * f_in, h), lambda j: (0, 0))],
        out_specs=pl.BlockSpec((blk, LANES), lambda j: (j, 0)),
        out_shape=jax.ShapeDtypeStruct((e_total, LANES), F32),
    )(ea_t, xs, s1, r1m, v1)

    agg1 = _sc_scatter_add(m1, dst2d, n)

    h1p, r2 = pl.pallas_call(
        _hidden_body,
        out_shape=[jax.ShapeDtypeStruct((n, LANES), F32),
                   jax.ShapeDtypeStruct((n, h), F32)],
    )(agg1, r1, root2, bias2.reshape(1, h))

    h1s = _sc_gather(h1p, src2d)

    m2 = pl.pallas_call(
        functools.partial(_msg_body, f_in=h),
        grid=(nblk,),
        in_specs=[pl.BlockSpec((dd, blk), lambda j: (0, j)),
                  pl.BlockSpec((blk, LANES), lambda j: (j, 0)),
                  pl.BlockSpec((dd, dd * h), lambda j: (0, 0)),
                  pl.BlockSpec((h, dd * h), lambda j: (0, 0)),
                  pl.BlockSpec((dd * h, h), lambda j: (0, 0))],
        out_specs=pl.BlockSpec((blk, LANES), lambda j: (j, 0)),
        out_shape=jax.ShapeDtypeStruct((e_total, LANES), F32),
    )(ea_t, h1s, s2, r2m, v2)

    agg2 = _sc_scatter_add(m2, dst2d, n)

    nb = 8
    bn = n // nb
    out = pl.pallas_call(
        functools.partial(_pool_body, n_graphs=N_GRAPHS, nblocks=nb),
        grid=(nb,),
        in_specs=[pl.BlockSpec((2, bn, LANES), lambda j: (0, j, 0)),
                  pl.BlockSpec((bn, h), lambda j: (j, 0)),
                  pl.BlockSpec((h, p_ch), lambda j: (0, 0)),
                  pl.BlockSpec((1, p_ch), lambda j: (0, 0)),
                  pl.BlockSpec((h, p_ch), lambda j: (0, 0)),
                  pl.BlockSpec((1, p_ch), lambda j: (0, 0)),
                  pl.BlockSpec((p_ch, n_out), lambda j: (0, 0)),
                  pl.BlockSpec((1, n_out), lambda j: (0, 0)),
                  pl.BlockSpec((1, bn), lambda j: (0, j))],
        out_specs=pl.BlockSpec((N_GRAPHS, n_out), lambda j: (0, 0)),
        out_shape=jax.ShapeDtypeStruct((N_GRAPHS, n_out), F32),
        scratch_shapes=[pltpu.VMEM((N_GRAPHS, p_ch), F32)],
    )(agg2, r2, Wf, bf.reshape(1, p_ch), Wa, ba.reshape(1, p_ch),
      Wd, bd.reshape(1, n_out), seg)
    return out


# speedup vs baseline: 5.3410x; 1.0041x over previous
import functools

import jax
import jax.numpy as jnp
from jax import lax
from jax.experimental import pallas as pl
from jax.experimental.pallas import tpu as pltpu
from jax.experimental.pallas import tpu_sc as plsc

F32 = jnp.float32
N_GRAPHS = 256
IDXBLK = 128
LANES = 128


def _pre_body(x_ref, scale_ref, beta_ref, root1_ref, bias1_ref, xp_ref, r1_ref):
    xn = x_ref[...] * scale_ref[...] + beta_ref[...]
    xp_ref[...] = jnp.concatenate(
        [xn, jnp.zeros((xn.shape[0], LANES - xn.shape[1]), F32)], axis=1)
    r1_ref[...] = jnp.dot(xn, root1_ref[...], preferred_element_type=F32) + bias1_ref[...]


def _msg_body(ea_ref, xs_ref, s_ref, r_ref, v_ref, out_ref, *, f_in):
    xn = xs_ref[:, :f_in]
    e7 = lax.dot_general(ea_ref[...], s_ref[...], (((0,), (0,)), ((), ())),
                         preferred_element_type=F32)
    z = e7 * jnp.dot(xn, r_ref[...], preferred_element_type=F32)
    m = jnp.dot(z, v_ref[...], preferred_element_type=F32)
    out_ref[...] = jnp.concatenate(
        [m, jnp.zeros((m.shape[0], LANES - m.shape[1]), F32)], axis=1)


def _hidden_body(agg_ref, r1_ref, root2_ref, bias2_ref, h1_ref, r2_ref):
    h = r1_ref.shape[1]
    h1 = jnp.maximum(agg_ref[0][:, :h] + agg_ref[1][:, :h] + r1_ref[...], 0.0)
    h1_ref[...] = jnp.concatenate(
        [h1, jnp.zeros((h1.shape[0], LANES - h), F32)], axis=1)
    r2_ref[...] = jnp.dot(h1, root2_ref[...], preferred_element_type=F32) + bias2_ref[...]


def _pool_body(agg_ref, r2_ref, wf_ref, bf_ref, wa_ref, ba_ref, wd_ref, bd_ref,
               seg_ref, out_ref, acc_ref, *, n_graphs, nblocks):
    j = pl.program_id(0)
    h = r2_ref.shape[1]
    h2 = jnp.maximum(agg_ref[0][:, :h] + agg_ref[1][:, :h] + r2_ref[...], 0.0)
    feat = jnp.dot(h2, wf_ref[...], preferred_element_type=F32) + bf_ref[...]
    attn = jax.nn.sigmoid(jnp.dot(h2, wa_ref[...], preferred_element_type=F32) + ba_ref[...])
    p = feat * attn
    seg = seg_ref[...]
    onehot = (seg == lax.broadcasted_iota(jnp.int32, (n_graphs, seg.shape[1]), 0)).astype(F32)
    part = jnp.dot(onehot, p, preferred_element_type=F32)

    @pl.when(j == 0)
    def _():
        acc_ref[...] = part

    @pl.when(j > 0)
    def _():
        acc_ref[...] = acc_ref[...] + part

    @pl.when(j == nblocks - 1)
    def _():
        out_ref[...] = (jnp.dot(acc_ref[...], wd_ref[...], preferred_element_type=F32)
                        + bd_ref[...])


def _sc_gather(table, idx2d):
    nrows_idx, _ = idx2d.shape
    e_total = nrows_idx * IDXBLK
    info = plsc.get_sparse_core_info()
    nc, ns = info.num_cores, info.num_subcores
    nw = nc * ns
    chunk = e_total // nw
    kblk = chunk // IDXBLK
    half = chunk // 2
    khalf = kblk // 2
    mesh = plsc.VectorSubcoreMesh(core_axis_name="c", subcore_axis_name="s")

    @functools.partial(
        pl.kernel,
        out_type=jax.ShapeDtypeStruct((e_total, LANES), F32),
        mesh=mesh,
        scratch_types=[
            pltpu.VMEM((kblk, IDXBLK), jnp.int32),
            pltpu.VMEM((half, LANES), F32),
            pltpu.SemaphoreType.DMA,
        ],
    )
    def gk(table_hbm, idx_hbm, out_hbm, idx_v, rows_v, sem):
        c = lax.axis_index("c")
        s = lax.axis_index("s")
        w = s * nc + c
        pltpu.sync_copy(idx_hbm.at[pl.ds(w * kblk, kblk)], idx_v)
        for hf in range(2):
            copies = []
            for j in range(khalf):
                copies.append(pltpu.async_copy(
                    table_hbm.at[idx_v.at[hf * khalf + j]],
                    rows_v.at[pl.ds(j * IDXBLK, IDXBLK)], sem))
            for cp in copies:
                cp.wait()
            pltpu.sync_copy(rows_v, out_hbm.at[pl.ds(w * chunk + hf * half, half)])

    return gk(table, idx2d)


def _sc_scatter_add(vals, idx2d, n_nodes):
    nrows_idx, _ = idx2d.shape
    e_total = nrows_idx * IDXBLK
    info = plsc.get_sparse_core_info()
    nc, ns = info.num_cores, info.num_subcores
    nw = nc * ns
    chunk = e_total // nw
    kblk = chunk // IDXBLK
    nparts = 4
    part = chunk // nparts
    kpart = kblk // nparts
    rows_per_tile = n_nodes // ns
    mesh = plsc.VectorSubcoreMesh(core_axis_name="c", subcore_axis_name="s")

    zrows = 16

    @functools.partial(
        pl.kernel,
        out_type=jax.ShapeDtypeStruct((nc, n_nodes, LANES), F32),
        mesh=mesh,
        scratch_types=[
            pltpu.VMEM((kblk, IDXBLK), jnp.int32),
            pltpu.VMEM((part, LANES), F32),
            pltpu.VMEM((zrows, LANES), F32),
            pltpu.VMEM_SHARED((n_nodes, LANES), F32),
            pltpu.SemaphoreType.DMA,
        ],
    )
    def sk(vals_hbm, idx_hbm, out_hbm, idx_v, vals_v, zbuf, acc_sh, sem):
        c = lax.axis_index("c")
        s = lax.axis_index("s")
        w = s * nc + c
        r0 = s * rows_per_tile
        nlane16 = LANES // 16

        def bz(k, _):
            zbuf[k // nlane16, pl.ds((k % nlane16) * 16, 16)] = jnp.zeros((16,), F32)
            return 0

        lax.fori_loop(0, zrows * nlane16, bz, 0)
        for t in range(rows_per_tile // zrows):
            pltpu.sync_copy(zbuf, acc_sh.at[pl.ds(r0 + t * zrows, zrows)])
        plsc.subcore_barrier()
        pltpu.sync_copy(idx_hbm.at[pl.ds(w * kblk, kblk)], idx_v)
        for hf in range(nparts):
            pltpu.sync_copy(vals_hbm.at[pl.ds(w * chunk + hf * part, part)], vals_v)
            for j in range(kpart):
                pltpu.sync_copy(vals_v.at[pl.ds(j * IDXBLK, IDXBLK)],
                                acc_sh.at[idx_v.at[hf * kpart + j]], add=True)
        plsc.subcore_barrier()
        pltpu.sync_copy(acc_sh.at[pl.ds(r0, rows_per_tile)],
                        out_hbm.at[c, pl.ds(r0, rows_per_tile)])

    return sk(vals, idx2d)


def kernel(x, e, gamma, beta, W1e, b1e, root1, bias1, W2e, b2e, root2, bias2,
           Wf, bf, Wa, ba, Wd, bd, edge_index, i):
    n, f_in = x.shape
    e_total, d_edge = e.shape
    h = root1.shape[1]
    p_ch = Wf.shape[1]
    n_out = Wd.shape[1]

    scale = (gamma * lax.rsqrt(jnp.float32(1.0 + 1e-3))).reshape(1, f_in)
    beta2 = beta.reshape(1, f_in)
    src2d = edge_index[0].reshape(e_total // IDXBLK, IDXBLK)
    dst2d = edge_index[1].reshape(e_total // IDXBLK, IDXBLK)
    dd = d_edge + 1
    ea_t = jnp.concatenate([e.T, jnp.ones((1, e_total), F32)], axis=0)
    v1 = jnp.concatenate([W1e, b1e[None, :]], axis=0).reshape(dd * f_in, h)
    v2 = jnp.concatenate([W2e, b2e[None, :]], axis=0).reshape(dd * h, h)
    s1 = jnp.kron(jnp.eye(dd, dtype=F32), jnp.ones((1, f_in), F32))
    r1m = jnp.tile(jnp.eye(f_in, dtype=F32), (1, dd))
    s2 = jnp.kron(jnp.eye(dd, dtype=F32), jnp.ones((1, h), F32))
    r2m = jnp.tile(jnp.eye(h, dtype=F32), (1, dd))
    seg = i.reshape(1, n)

    xp, r1 = pl.pallas_call(
        _pre_body,
        out_shape=[jax.ShapeDtypeStruct((n, LANES), F32),
                   jax.ShapeDtypeStruct((n, h), F32)],
    )(x, scale, beta2, root1, bias1.reshape(1, h))

    xs = _sc_gather(xp, src2d)

    blk = 4096
    nblk = e_total // blk
    m1 = pl.pallas_call(
        functools.partial(_msg_body, f_in=f_in),
        grid=(nblk,),
        in_specs=[pl.BlockSpec((dd, blk), lambda j: (0, j)),
                  pl.BlockSpec((blk, LANES), lambda j: (j, 0)),
                  pl.BlockSpec((dd, dd * f_in), lambda j: (0, 0)),
                  pl.BlockSpec((f_in, dd * f_in), lambda j: (0, 0)),
                  pl.BlockSpec((dd * f_in, h), lambda j: (0, 0))],
        out_specs=pl.BlockSpec((blk, LANES), lambda j: (j, 0)),
        out_shape=jax.ShapeDtypeStruct((e_total, LANES), F32),
    )(ea_t, xs, s1, r1m, v1)

    agg1 = _sc_scatter_add(m1, dst2d, n)

    h1p, r2 = pl.pallas_call(
        _hidden_body,
        out_shape=[jax.ShapeDtypeStruct((n, LANES), F32),
                   jax.ShapeDtypeStruct((n, h), F32)],
    )(agg1, r1, root2, bias2.reshape(1, h))

    h1s = _sc_gather(h1p, src2d)

    m2 = pl.pallas_call(
        functools.partial(_msg_body, f_in=h),
        grid=(nblk,),
        in_specs=[pl.BlockSpec((dd, blk), lambda j: (0, j)),
                  pl.BlockSpec((blk, LANES), lambda j: (j, 0)),
                  pl.BlockSpec((dd, dd * h), lambda j: (0, 0)),
                  pl.BlockSpec((h, dd * h), lambda j: (0, 0)),
                  pl.BlockSpec((dd * h, h), lambda j: (0, 0))],
        out_specs=pl.BlockSpec((blk, LANES), lambda j: (j, 0)),
        out_shape=jax.ShapeDtypeStruct((e_total, LANES), F32),
    )(ea_t, h1s, s2, r2m, v2)

    agg2 = _sc_scatter_add(m2, dst2d, n)

    nb = 8
    bn = n // nb
    out = pl.pallas_call(
        functools.partial(_pool_body, n_graphs=N_GRAPHS, nblocks=nb),
        grid=(nb,),
        in_specs=[pl.BlockSpec((2, bn, LANES), lambda j: (0, j, 0)),
                  pl.BlockSpec((bn, h), lambda j: (j, 0)),
                  pl.BlockSpec((h, p_ch), lambda j: (0, 0)),
                  pl.BlockSpec((1, p_ch), lambda j: (0, 0)),
                  pl.BlockSpec((h, p_ch), lambda j: (0, 0)),
                  pl.BlockSpec((1, p_ch), lambda j: (0, 0)),
                  pl.BlockSpec((p_ch, n_out), lambda j: (0, 0)),
                  pl.BlockSpec((1, n_out), lambda j: (0, 0)),
                  pl.BlockSpec((1, bn), lambda j: (0, j))],
        out_specs=pl.BlockSpec((N_GRAPHS, n_out), lambda j: (0, 0)),
        out_shape=jax.ShapeDtypeStruct((N_GRAPHS, n_out), F32),
        scratch_shapes=[pltpu.VMEM((N_GRAPHS, p_ch), F32)],
    )(agg2, r2, Wf, bf.reshape(1, p_ch), Wa, ba.reshape(1, p_ch),
      Wd, bd.reshape(1, n_out), seg)
    return out


# speedup vs baseline: 5.6715x; 1.0619x over previous
import functools

import jax
import jax.numpy as jnp
from jax import lax
from jax.experimental import pallas as pl
from jax.experimental.pallas import tpu as pltpu
from jax.experimental.pallas import tpu_sc as plsc

F32 = jnp.float32
N_GRAPHS = 256
IDXBLK = 128
LANES = 128


def _pre_body(x_ref, scale_ref, beta_ref, root1_ref, bias1_ref, xp_ref, r1_ref):
    xn = x_ref[...] * scale_ref[...] + beta_ref[...]
    xp_ref[...] = jnp.concatenate(
        [xn, jnp.zeros((xn.shape[0], LANES - xn.shape[1]), F32)], axis=1)
    r1_ref[...] = jnp.dot(xn, root1_ref[...], preferred_element_type=F32) + bias1_ref[...]


def _msg_body(ea_ref, xs_ref, s_ref, r_ref, v_ref, out_ref, *, f_in):
    xn = xs_ref[:, :f_in]
    e7 = lax.dot_general(ea_ref[...], s_ref[...], (((0,), (0,)), ((), ())),
                         preferred_element_type=F32)
    z = e7 * jnp.dot(xn, r_ref[...], preferred_element_type=F32)
    m = jnp.dot(z, v_ref[...], preferred_element_type=F32)
    out_ref[...] = jnp.concatenate(
        [m, jnp.zeros((m.shape[0], LANES - m.shape[1]), F32)], axis=1)


def _hidden_body(agg_ref, r1_ref, root2_ref, bias2_ref, h1_ref, r2_ref):
    h = r1_ref.shape[1]
    h1 = jnp.maximum(agg_ref[0][:, :h] + agg_ref[1][:, :h] + r1_ref[...], 0.0)
    h1_ref[...] = jnp.concatenate(
        [h1, jnp.zeros((h1.shape[0], LANES - h), F32)], axis=1)
    r2_ref[...] = jnp.dot(h1, root2_ref[...], preferred_element_type=F32) + bias2_ref[...]


def _pool_body(agg_ref, r2_ref, wf_ref, bf_ref, wa_ref, ba_ref, wd_ref, bd_ref,
               seg_ref, out_ref, acc_ref, *, n_graphs, nblocks):
    j = pl.program_id(0)
    h = r2_ref.shape[1]
    h2 = jnp.maximum(agg_ref[0][:, :h] + agg_ref[1][:, :h] + r2_ref[...], 0.0)
    feat = jnp.dot(h2, wf_ref[...], preferred_element_type=F32) + bf_ref[...]
    attn = jax.nn.sigmoid(jnp.dot(h2, wa_ref[...], preferred_element_type=F32) + ba_ref[...])
    p = feat * attn
    seg = seg_ref[...]
    onehot = (seg == lax.broadcasted_iota(jnp.int32, (n_graphs, seg.shape[1]), 0)).astype(F32)
    part = jnp.dot(onehot, p, preferred_element_type=F32)

    @pl.when(j == 0)
    def _():
        acc_ref[...] = part

    @pl.when(j > 0)
    def _():
        acc_ref[...] = acc_ref[...] + part

    @pl.when(j == nblocks - 1)
    def _():
        out_ref[...] = (jnp.dot(acc_ref[...], wd_ref[...], preferred_element_type=F32)
                        + bd_ref[...])


def _sc_gather(table, idx2d):
    nrows_idx, _ = idx2d.shape
    e_total = nrows_idx * IDXBLK
    info = plsc.get_sparse_core_info()
    nc, ns = info.num_cores, info.num_subcores
    nw = nc * ns
    chunk = e_total // nw
    kblk = chunk // IDXBLK
    half = chunk // 2
    khalf = kblk // 2
    mesh = plsc.VectorSubcoreMesh(core_axis_name="c", subcore_axis_name="s")

    nparts = 4
    kpart = kblk // nparts
    prows = chunk // nparts
    nbuf = 2

    @functools.partial(
        pl.kernel,
        out_type=jax.ShapeDtypeStruct((e_total, LANES), F32),
        mesh=mesh,
        scratch_types=[
            pltpu.VMEM((kblk, IDXBLK), jnp.int32),
            [pltpu.VMEM((prows, LANES), F32) for _ in range(nbuf)],
            [pltpu.SemaphoreType.DMA for _ in range(nbuf)],
            [pltpu.SemaphoreType.DMA for _ in range(nbuf)],
        ],
    )
    def gk(table_hbm, idx_hbm, out_hbm, idx_v, bufs, gsems, osems):
        c = lax.axis_index("c")
        s = lax.axis_index("s")
        w = s * nc + c
        pltpu.sync_copy(idx_hbm.at[pl.ds(w * kblk, kblk)], idx_v)
        outcp = [None] * nbuf
        for p in range(nparts):
            b = p % nbuf
            if outcp[b] is not None:
                outcp[b].wait()
            cps = []
            for j in range(kpart):
                cps.append(pltpu.async_copy(
                    table_hbm.at[idx_v.at[p * kpart + j]],
                    bufs[b].at[pl.ds(j * IDXBLK, IDXBLK)], gsems[b]))
            for cp in cps:
                cp.wait()
            outcp[b] = pltpu.async_copy(
                bufs[b], out_hbm.at[pl.ds(w * chunk + p * prows, prows)], osems[b])
        for cp in outcp:
            cp.wait()

    return gk(table, idx2d)


def _sc_scatter_add(vals, idx2d, n_nodes):
    nrows_idx, _ = idx2d.shape
    e_total = nrows_idx * IDXBLK
    info = plsc.get_sparse_core_info()
    nc, ns = info.num_cores, info.num_subcores
    nw = nc * ns
    chunk = e_total // nw
    kblk = chunk // IDXBLK
    nparts = kblk
    rows_per_tile = n_nodes // ns
    mesh = plsc.VectorSubcoreMesh(core_axis_name="c", subcore_axis_name="s")

    zrows = 16
    nbuf = 3

    @functools.partial(
        pl.kernel,
        out_type=jax.ShapeDtypeStruct((nc, n_nodes, LANES), F32),
        mesh=mesh,
        scratch_types=[
            pltpu.VMEM((kblk, IDXBLK), jnp.int32),
            [pltpu.VMEM((IDXBLK, LANES), F32) for _ in range(nbuf)],
            pltpu.VMEM((zrows, LANES), F32),
            pltpu.VMEM_SHARED((n_nodes, LANES), F32),
            [pltpu.SemaphoreType.DMA for _ in range(nbuf)],
            [pltpu.SemaphoreType.DMA for _ in range(nbuf)],
        ],
    )
    def sk(vals_hbm, idx_hbm, out_hbm, idx_v, bufs, zbuf, acc_sh, lsems, asems):
        c = lax.axis_index("c")
        s = lax.axis_index("s")
        w = s * nc + c
        r0 = s * rows_per_tile
        pltpu.sync_copy(idx_hbm.at[pl.ds(w * kblk, kblk)], idx_v)
        loadcp = [None] * nparts
        for p in range(nbuf):
            loadcp[p] = pltpu.async_copy(
                vals_hbm.at[pl.ds(w * chunk + p * IDXBLK, IDXBLK)],
                bufs[p], lsems[p])
        nlane16 = LANES // 16

        def bz(k, _):
            zbuf[k // nlane16, pl.ds((k % nlane16) * 16, 16)] = jnp.zeros((16,), F32)
            return 0

        lax.fori_loop(0, zrows * nlane16, bz, 0)
        for t in range(rows_per_tile // zrows):
            pltpu.sync_copy(zbuf, acc_sh.at[pl.ds(r0 + t * zrows, zrows)])
        plsc.subcore_barrier()
        addcp = [None] * nparts
        for p in range(nparts):
            b = p % nbuf
            loadcp[p].wait()
            addcp[p] = pltpu.async_copy(bufs[b], acc_sh.at[idx_v.at[p]],
                                        asems[b], add=True)
            q = p + nbuf
            if q < nparts:
                addcp[p].wait()
                loadcp[q] = pltpu.async_copy(
                    vals_hbm.at[pl.ds(w * chunk + q * IDXBLK, IDXBLK)],
                    bufs[b], lsems[b])
        for p in range(nparts - nbuf, nparts):
            addcp[p].wait()
        plsc.subcore_barrier()
        pltpu.sync_copy(acc_sh.at[pl.ds(r0, rows_per_tile)],
                        out_hbm.at[c, pl.ds(r0, rows_per_tile)])

    return sk(vals, idx2d)


def kernel(x, e, gamma, beta, W1e, b1e, root1, bias1, W2e, b2e, root2, bias2,
           Wf, bf, Wa, ba, Wd, bd, edge_index, i):
    n, f_in = x.shape
    e_total, d_edge = e.shape
    h = root1.shape[1]
    p_ch = Wf.shape[1]
    n_out = Wd.shape[1]

    scale = (gamma * lax.rsqrt(jnp.float32(1.0 + 1e-3))).reshape(1, f_in)
    beta2 = beta.reshape(1, f_in)
    src2d = edge_index[0].reshape(e_total // IDXBLK, IDXBLK)
    dst2d = edge_index[1].reshape(e_total // IDXBLK, IDXBLK)
    dd = d_edge + 1
    ea_t = jnp.concatenate([e.T, jnp.ones((1, e_total), F32)], axis=0)
    v1 = jnp.concatenate([W1e, b1e[None, :]], axis=0).reshape(dd * f_in, h)
    v2 = jnp.concatenate([W2e, b2e[None, :]], axis=0).reshape(dd * h, h)
    s1 = jnp.kron(jnp.eye(dd, dtype=F32), jnp.ones((1, f_in), F32))
    r1m = jnp.tile(jnp.eye(f_in, dtype=F32), (1, dd))
    s2 = jnp.kron(jnp.eye(dd, dtype=F32), jnp.ones((1, h), F32))
    r2m = jnp.tile(jnp.eye(h, dtype=F32), (1, dd))
    seg = i.reshape(1, n)

    xp, r1 = pl.pallas_call(
        _pre_body,
        out_shape=[jax.ShapeDtypeStruct((n, LANES), F32),
                   jax.ShapeDtypeStruct((n, h), F32)],
    )(x, scale, beta2, root1, bias1.reshape(1, h))

    xs = _sc_gather(xp, src2d)

    blk = 4096
    nblk = e_total // blk
    m1 = pl.pallas_call(
        functools.partial(_msg_body, f_in=f_in),
        grid=(nblk,),
        in_specs=[pl.BlockSpec((dd, blk), lambda j: (0, j)),
                  pl.BlockSpec((blk, LANES), lambda j: (j, 0)),
                  pl.BlockSpec((dd, dd * f_in), lambda j: (0, 0)),
                  pl.BlockSpec((f_in, dd * f_in), lambda j: (0, 0)),
                  pl.BlockSpec((dd * f_in, h), lambda j: (0, 0))],
        out_specs=pl.BlockSpec((blk, LANES), lambda j: (j, 0)),
        out_shape=jax.ShapeDtypeStruct((e_total, LANES), F32),
    )(ea_t, xs, s1, r1m, v1)

    agg1 = _sc_scatter_add(m1, dst2d, n)

    h1p, r2 = pl.pallas_call(
        _hidden_body,
        out_shape=[jax.ShapeDtypeStruct((n, LANES), F32),
                   jax.ShapeDtypeStruct((n, h), F32)],
    )(agg1, r1, root2, bias2.reshape(1, h))

    h1s = _sc_gather(h1p, src2d)

    m2 = pl.pallas_call(
        functools.partial(_msg_body, f_in=h),
        grid=(nblk,),
        in_specs=[pl.BlockSpec((dd, blk), lambda j: (0, j)),
                  pl.BlockSpec((blk, LANES), lambda j: (j, 0)),
                  pl.BlockSpec((dd, dd * h), lambda j: (0, 0)),
                  pl.BlockSpec((h, dd * h), lambda j: (0, 0)),
                  pl.BlockSpec((dd * h, h), lambda j: (0, 0))],
        out_specs=pl.BlockSpec((blk, LANES), lambda j: (j, 0)),
        out_shape=jax.ShapeDtypeStruct((e_total, LANES), F32),
    )(ea_t, h1s, s2, r2m, v2)

    agg2 = _sc_scatter_add(m2, dst2d, n)

    nb = 8
    bn = n // nb
    out = pl.pallas_call(
        functools.partial(_pool_body, n_graphs=N_GRAPHS, nblocks=nb),
        grid=(nb,),
        in_specs=[pl.BlockSpec((2, bn, LANES), lambda j: (0, j, 0)),
                  pl.BlockSpec((bn, h), lambda j: (j, 0)),
                  pl.BlockSpec((h, p_ch), lambda j: (0, 0)),
                  pl.BlockSpec((1, p_ch), lambda j: (0, 0)),
                  pl.BlockSpec((h, p_ch), lambda j: (0, 0)),
                  pl.BlockSpec((1, p_ch), lambda j: (0, 0)),
                  pl.BlockSpec((p_ch, n_out), lambda j: (0, 0)),
                  pl.BlockSpec((1, n_out), lambda j: (0, 0)),
                  pl.BlockSpec((1, bn), lambda j: (0, j))],
        out_specs=pl.BlockSpec((N_GRAPHS, n_out), lambda j: (0, 0)),
        out_shape=jax.ShapeDtypeStruct((N_GRAPHS, n_out), F32),
        scratch_shapes=[pltpu.VMEM((N_GRAPHS, p_ch), F32)],
    )(agg2, r2, Wf, bf.reshape(1, p_ch), Wa, ba.reshape(1, p_ch),
      Wd, bd.reshape(1, n_out), seg)
    return out
